# baseline (speedup 1.0000x reference)
import functools

import jax
import jax.numpy as jnp
from jax import lax
from jax.experimental import pallas as pl
from jax.experimental.pallas import tpu as pltpu
from jax.experimental.pallas import tpu_sc as plsc

N_NODES = 10000
F = 128
NC, NS, L = 2, 16, 16
NW = NC * NS
N_PAD = 10240
RPT = N_PAD // NS
CHUNK = 64
DEG_W = 16
ZB = 16
IBATCH = 40
DEPTH = 3
ROW_BLK = 2000


def _mesh():
    return plsc.VectorSubcoreMesh(core_axis_name="c", subcore_axis_name="s")


def _deg_body(nch_w, dst_hbm_eidx, degp_hbm, ibuf, hist, sumb, res, slab):
    c = lax.axis_index("c")
    s = lax.axis_index("s")
    wid = c * NS + s
    w0 = wid * nch_w
    zeros16 = jnp.zeros((L,), jnp.float32)
    ones16 = jnp.ones((L,), jnp.float32)

    def zfill(i, carry):
        hist[pl.ds(i * L, L)] = zeros16
        return carry

    lax.fori_loop(0, N_PAD // L, zfill, 0)

    nb = nch_w // IBATCH
    for b in range(nb):
        pltpu.sync_copy(dst_hbm_eidx.at[pl.ds(w0 + b * IBATCH, IBATCH)], ibuf)
        for j in range(IBATCH):
            for k in range(CHUNK // L):
                idx = ibuf[j, 1, pl.ds(k * L, L)]
                plsc.addupdate_scatter(hist, [idx], ones16)

    pltpu.sync_copy(hist, slab.at[s])
    plsc.subcore_barrier()
    pltpu.sync_copy(slab.at[:, pl.ds(s * RPT, RPT)], sumb)

    def comb(k, carry):
        acc = sumb[0, pl.ds(k * L, L)]
        for t in range(1, NS):
            acc = acc + sumb[t, pl.ds(k * L, L)]
        idx = lax.iota(jnp.int32, L) * DEG_W + k * (L * DEG_W)
        plsc.store_scatter(res, [idx], acc)
        return carry

    lax.fori_loop(0, RPT // L, comb, 0)
    pltpu.sync_copy(res, degp_hbm.at[c, pl.ds(s * RPT * DEG_W, RPT * DEG_W)])


def _deg_call(eidx, nch_w):
    k = pl.kernel(
        functools.partial(_deg_body, nch_w),
        out_type=jax.ShapeDtypeStruct((NC, N_PAD * DEG_W), jnp.float32),
        mesh=_mesh(),
        compiler_params=pltpu.CompilerParams(needs_layout_passes=False),
        scratch_types=[
            pltpu.VMEM((IBATCH, 2, CHUNK), jnp.int32),
            pltpu.VMEM((N_PAD,), jnp.float32),
            pltpu.VMEM((NS, RPT), jnp.float32),
            pltpu.VMEM((RPT * DEG_W,), jnp.float32),
            pltpu.VMEM_SHARED((NS, N_PAD), jnp.float32),
        ],
    )
    return k(eidx).reshape(NC, N_PAD, DEG_W)


def _scat_body(nch_w, hs_hbm, eidx_hbm, part_hbm, ibuf, rows0, rows1, rows2,
               stage, acc, gsem0, gsem1, gsem2, isem):
    c = lax.axis_index("c")
    s = lax.axis_index("s")
    wid = c * NS + s
    w0 = wid * nch_w
    zeros16 = jnp.zeros((L,), jnp.float32)

    def zero(i, carry):
        for j in range(F // L):
            stage[i, pl.ds(j * L, L)] = zeros16
        return carry

    lax.fori_loop(0, ZB, zero, 0)
    row0 = s * RPT

    def zcopy(k, carry):
        pltpu.sync_copy(stage, acc.at[pl.ds(row0 + k * ZB, ZB)])
        return carry

    lax.fori_loop(0, RPT // ZB, zcopy, 0)
    plsc.subcore_barrier()

    rows = [rows0, rows1, rows2]
    gsem = [gsem0, gsem1, gsem2]
    nb = nch_w // IBATCH
    total = nb * IBATCH

    pltpu.sync_copy(eidx_hbm.at[pl.ds(w0, IBATCH)], ibuf.at[0])
    gathers = [None] * DEPTH
    for g0 in range(min(DEPTH - 1, total)):
        gathers[g0] = pltpu.async_copy(
            hs_hbm.at[ibuf.at[0, g0, 0]], rows[g0], gsem[g0])
    iload = None
    for g in range(total):
        b = g // IBATCH
        if g % IBATCH == 0 and b + 1 < nb:
            iload = pltpu.async_copy(
                eidx_hbm.at[pl.ds(w0 + (b + 1) * IBATCH, IBATCH)],
                ibuf.at[(b + 1) % 2], isem)
        nxt = g + DEPTH - 1
        if nxt < total:
            nbt, nj = divmod(nxt, IBATCH)
            if nj == 0:
                iload.wait()
            gathers[nxt % DEPTH] = pltpu.async_copy(
                hs_hbm.at[ibuf.at[nbt % 2, nj, 0]], rows[nxt % DEPTH],
                gsem[nxt % DEPTH])
        gathers[g % DEPTH].wait()
        pltpu.sync_copy(rows[g % DEPTH], acc.at[ibuf.at[b % 2, g % IBATCH, 1]],
                        add=True)

    plsc.subcore_barrier()

    def wcopy(k, carry):
        pltpu.sync_copy(acc.at[pl.ds(row0 + k * ZB, ZB)], stage)
        pltpu.sync_copy(stage, part_hbm.at[c, pl.ds(row0 + k * ZB, ZB)])
        return carry

    lax.fori_loop(0, RPT // ZB, wcopy, 0)


def _scat_call(hs, eidx, nch_w):
    k = pl.kernel(
        functools.partial(_scat_body, nch_w),
        out_type=jax.ShapeDtypeStruct((NC, N_PAD, F), jnp.float32),
        mesh=_mesh(),
        scratch_types=[
            pltpu.VMEM((2, IBATCH, 2, CHUNK), jnp.int32),
            pltpu.VMEM((CHUNK, F), jnp.float32),
            pltpu.VMEM((CHUNK, F), jnp.float32),
            pltpu.VMEM((CHUNK, F), jnp.float32),
            pltpu.VMEM((ZB, F), jnp.float32),
            pltpu.VMEM_SHARED((N_PAD, F), jnp.float32),
            pltpu.SemaphoreType.DMA,
            pltpu.SemaphoreType.DMA,
            pltpu.SemaphoreType.DMA,
            pltpu.SemaphoreType.DMA,
        ],
    )
    return k(hs, eidx)


def _mm_body(x_ref, w_ref, degp_ref, hs_ref):
    deg = degp_ref[0, :, 0:1] + degp_ref[1, :, 0:1] + 1.0
    dis = lax.rsqrt(deg)
    h = jnp.dot(x_ref[...], w_ref[...], preferred_element_type=jnp.float32)
    hs_ref[...] = h * dis


def _mm_call(x, W, degp):
    grid = (N_NODES // ROW_BLK,)
    return pl.pallas_call(
        _mm_body,
        grid=grid,
        in_specs=[
            pl.BlockSpec((ROW_BLK, F), lambda i: (i, 0)),
            pl.BlockSpec((F, F), lambda i: (0, 0)),
            pl.BlockSpec((NC, ROW_BLK, DEG_W), lambda i: (0, i, 0)),
        ],
        out_specs=pl.BlockSpec((ROW_BLK, F), lambda i: (i, 0)),
        out_shape=jax.ShapeDtypeStruct((N_PAD, F), jnp.float32),
    )(x, W, degp)


def _fin_body(part_ref, hs_ref, degp_ref, b_ref, out_ref):
    deg = degp_ref[0, :, 0:1] + degp_ref[1, :, 0:1] + 1.0
    dis = lax.rsqrt(deg)
    out_ref[...] = dis * (part_ref[0] + part_ref[1] + hs_ref[...]) + b_ref[...]


def _fin_call(part, hs, degp, b2d):
    grid = (N_NODES // ROW_BLK,)
    return pl.pallas_call(
        _fin_body,
        grid=grid,
        in_specs=[
            pl.BlockSpec((NC, ROW_BLK, F), lambda i: (0, i, 0)),
            pl.BlockSpec((ROW_BLK, F), lambda i: (i, 0)),
            pl.BlockSpec((NC, ROW_BLK, DEG_W), lambda i: (0, i, 0)),
            pl.BlockSpec((1, F), lambda i: (0, 0)),
        ],
        out_specs=pl.BlockSpec((ROW_BLK, F), lambda i: (i, 0)),
        out_shape=jax.ShapeDtypeStruct((N_NODES, F), jnp.float32),
    )(part, hs, degp, b2d)


def kernel(x, edge_index, W, b):
    n, f_in = x.shape
    e = edge_index.shape[1]
    assert n == N_NODES and f_in == F

    nch = -(-e // CHUNK)
    nch_w = -(-nch // NW)
    nch_w = -(-nch_w // IBATCH) * IBATCH
    e_pad = nch_w * NW * CHUNK
    fill = N_NODES + (jnp.arange(e_pad - e, dtype=jnp.int32)
                      % (N_PAD - N_NODES))
    ep = jnp.concatenate(
        [edge_index.astype(jnp.int32), jnp.stack([fill, fill])], axis=1)
    eidx = ep.reshape(2, nch_w * NW, CHUNK).transpose(1, 0, 2)

    degp = _deg_call(eidx, nch_w)
    hs = _mm_call(x, W, degp)
    part = _scat_call(hs, eidx, nch_w)
    out = _fin_call(part, hs, degp, b.reshape(1, F))
    return out


# speedup vs baseline: 1.0826x; 1.0826x over previous
import functools

import jax
import jax.numpy as jnp
from jax import lax
from jax.experimental import pallas as pl
from jax.experimental.pallas import tpu as pltpu
from jax.experimental.pallas import tpu_sc as plsc

N_NODES = 10000
F = 128
NC, NS, L = 2, 16, 16
NW = NC * NS
N_PAD = 10240
RPT = N_PAD // NS
CHUNK = 64
DEG_W = 16
ZB = 16
IBATCH = 40
DEPTH = 3
ROW_BLK = 2000


def _mesh():
    return plsc.VectorSubcoreMesh(core_axis_name="c", subcore_axis_name="s")


def _deg_body(nch_w, dst_hbm, degp_hbm, ibuf, hist, sumb, res, slab):
    c = lax.axis_index("c")
    s = lax.axis_index("s")
    wid = c * NS + s
    w0 = wid * nch_w
    zeros16 = jnp.zeros((L,), jnp.float32)
    ones16 = jnp.ones((L,), jnp.float32)

    def zfill(i, carry):
        hist[pl.ds(i * L, L)] = zeros16
        return carry

    lax.fori_loop(0, N_PAD // L, zfill, 0)

    nb = nch_w // IBATCH
    for b in range(nb):
        pltpu.sync_copy(dst_hbm.at[pl.ds(w0 + b * IBATCH, IBATCH)], ibuf)
        for j in range(IBATCH):
            for k in range(CHUNK // L):
                idx = ibuf[j, pl.ds(k * L, L)]
                plsc.addupdate_scatter(hist, [idx], ones16)

    pltpu.sync_copy(hist, slab.at[s])
    plsc.subcore_barrier()
    pltpu.sync_copy(slab.at[:, pl.ds(s * RPT, RPT)], sumb)

    def comb(k, carry):
        acc = sumb[0, pl.ds(k * L, L)]
        for t in range(1, NS):
            acc = acc + sumb[t, pl.ds(k * L, L)]
        idx = lax.iota(jnp.int32, L) * DEG_W + k * (L * DEG_W)
        plsc.store_scatter(res, [idx], acc)
        return carry

    lax.fori_loop(0, RPT // L, comb, 0)
    pltpu.sync_copy(res, degp_hbm.at[c, pl.ds(s * RPT * DEG_W, RPT * DEG_W)])


def _deg_call(eidx, nch_w):
    k = pl.kernel(
        functools.partial(_deg_body, nch_w),
        out_type=jax.ShapeDtypeStruct((NC, N_PAD * DEG_W), jnp.float32),
        mesh=_mesh(),
        compiler_params=pltpu.CompilerParams(needs_layout_passes=False),
        scratch_types=[
            pltpu.VMEM((IBATCH, CHUNK), jnp.int32),
            pltpu.VMEM((N_PAD,), jnp.float32),
            pltpu.VMEM((NS, RPT), jnp.float32),
            pltpu.VMEM((RPT * DEG_W,), jnp.float32),
            pltpu.VMEM_SHARED((NS, N_PAD), jnp.float32),
        ],
    )
    return k(eidx).reshape(NC, N_PAD, DEG_W)


def _scat_body(nch_w, hs_hbm, src_hbm, dst_hbm, part_hbm, ibs, ibd, rows0,
               rows1, rows2, stage, acc, gsem0, gsem1, gsem2, isem):
    c = lax.axis_index("c")
    s = lax.axis_index("s")
    wid = c * NS + s
    w0 = wid * nch_w
    zeros16 = jnp.zeros((L,), jnp.float32)

    def zero(i, carry):
        for j in range(F // L):
            stage[i, pl.ds(j * L, L)] = zeros16
        return carry

    lax.fori_loop(0, ZB, zero, 0)
    row0 = s * RPT

    def zcopy(k, carry):
        pltpu.sync_copy(stage, acc.at[pl.ds(row0 + k * ZB, ZB)])
        return carry

    lax.fori_loop(0, RPT // ZB, zcopy, 0)
    plsc.subcore_barrier()

    rows = [rows0, rows1, rows2]
    gsem = [gsem0, gsem1, gsem2]
    nb = nch_w // IBATCH
    total = nb * IBATCH

    pltpu.sync_copy(src_hbm.at[pl.ds(w0, IBATCH)], ibs.at[0])
    pltpu.sync_copy(dst_hbm.at[pl.ds(w0, IBATCH)], ibd.at[0])
    gathers = [None] * DEPTH
    for g0 in range(min(DEPTH - 1, total)):
        gathers[g0] = pltpu.async_copy(
            hs_hbm.at[ibs.at[0, g0]], rows[g0], gsem[g0])
    iload_s = iload_d = None
    for g in range(total):
        b = g // IBATCH
        if g % IBATCH == 0 and b + 1 < nb:
            iload_s = pltpu.async_copy(
                src_hbm.at[pl.ds(w0 + (b + 1) * IBATCH, IBATCH)],
                ibs.at[(b + 1) % 2], isem)
            iload_d = pltpu.async_copy(
                dst_hbm.at[pl.ds(w0 + (b + 1) * IBATCH, IBATCH)],
                ibd.at[(b + 1) % 2], isem)
        nxt = g + DEPTH - 1
        if nxt < total:
            nbt, nj = divmod(nxt, IBATCH)
            if nj == 0:
                iload_s.wait()
                iload_d.wait()
            gathers[nxt % DEPTH] = pltpu.async_copy(
                hs_hbm.at[ibs.at[nbt % 2, nj]], rows[nxt % DEPTH],
                gsem[nxt % DEPTH])
        gathers[g % DEPTH].wait()
        pltpu.sync_copy(rows[g % DEPTH], acc.at[ibd.at[b % 2, g % IBATCH]],
                        add=True)

    plsc.subcore_barrier()

    def wcopy(k, carry):
        pltpu.sync_copy(acc.at[pl.ds(row0 + k * ZB, ZB)], stage)
        pltpu.sync_copy(stage, part_hbm.at[c, pl.ds(row0 + k * ZB, ZB)])
        return carry

    lax.fori_loop(0, RPT // ZB, wcopy, 0)


def _scat_call(hs, src_r, dst_r, nch_w):
    k = pl.kernel(
        functools.partial(_scat_body, nch_w),
        out_type=jax.ShapeDtypeStruct((NC, N_PAD, F), jnp.float32),
        mesh=_mesh(),
        scratch_types=[
            pltpu.VMEM((2, IBATCH, CHUNK), jnp.int32),
            pltpu.VMEM((2, IBATCH, CHUNK), jnp.int32),
            pltpu.VMEM((CHUNK, F), jnp.float32),
            pltpu.VMEM((CHUNK, F), jnp.float32),
            pltpu.VMEM((CHUNK, F), jnp.float32),
            pltpu.VMEM((ZB, F), jnp.float32),
            pltpu.VMEM_SHARED((N_PAD, F), jnp.float32),
            pltpu.SemaphoreType.DMA,
            pltpu.SemaphoreType.DMA,
            pltpu.SemaphoreType.DMA,
            pltpu.SemaphoreType.DMA,
        ],
    )
    return k(hs, src_r, dst_r)


def _mm_body(x_ref, w_ref, degp_ref, hs_ref):
    deg = degp_ref[0, :, 0:1] + degp_ref[1, :, 0:1] + 1.0
    dis = lax.rsqrt(deg)
    h = jnp.dot(x_ref[...], w_ref[...], preferred_element_type=jnp.float32)
    hs_ref[...] = h * dis


def _mm_call(x, W, degp):
    grid = (N_NODES // ROW_BLK,)
    return pl.pallas_call(
        _mm_body,
        grid=grid,
        in_specs=[
            pl.BlockSpec((ROW_BLK, F), lambda i: (i, 0)),
            pl.BlockSpec((F, F), lambda i: (0, 0)),
            pl.BlockSpec((NC, ROW_BLK, DEG_W), lambda i: (0, i, 0)),
        ],
        out_specs=pl.BlockSpec((ROW_BLK, F), lambda i: (i, 0)),
        out_shape=jax.ShapeDtypeStruct((N_PAD, F), jnp.float32),
    )(x, W, degp)


def _fin_body(part_ref, hs_ref, degp_ref, b_ref, out_ref):
    deg = degp_ref[0, :, 0:1] + degp_ref[1, :, 0:1] + 1.0
    dis = lax.rsqrt(deg)
    out_ref[...] = dis * (part_ref[0] + part_ref[1] + hs_ref[...]) + b_ref[...]


def _fin_call(part, hs, degp, b2d):
    grid = (N_NODES // ROW_BLK,)
    return pl.pallas_call(
        _fin_body,
        grid=grid,
        in_specs=[
            pl.BlockSpec((NC, ROW_BLK, F), lambda i: (0, i, 0)),
            pl.BlockSpec((ROW_BLK, F), lambda i: (i, 0)),
            pl.BlockSpec((NC, ROW_BLK, DEG_W), lambda i: (0, i, 0)),
            pl.BlockSpec((1, F), lambda i: (0, 0)),
        ],
        out_specs=pl.BlockSpec((ROW_BLK, F), lambda i: (i, 0)),
        out_shape=jax.ShapeDtypeStruct((N_NODES, F), jnp.float32),
    )(part, hs, degp, b2d)


def kernel(x, edge_index, W, b):
    n, f_in = x.shape
    e = edge_index.shape[1]
    assert n == N_NODES and f_in == F

    nch = -(-e // CHUNK)
    nch_w = -(-nch // NW)
    nch_w = -(-nch_w // IBATCH) * IBATCH
    e_pad = nch_w * NW * CHUNK
    fill = N_NODES + (jnp.arange(e_pad - e, dtype=jnp.int32)
                      % (N_PAD - N_NODES))
    src_r = jnp.concatenate([edge_index[0].astype(jnp.int32), fill]
                            ).reshape(nch_w * NW, CHUNK)
    dst_r = jnp.concatenate([edge_index[1].astype(jnp.int32), fill]
                            ).reshape(nch_w * NW, CHUNK)

    degp = _deg_call(dst_r, nch_w)
    hs = _mm_call(x, W, degp)
    part = _scat_call(hs, src_r, dst_r, nch_w)
    out = _fin_call(part, hs, degp, b.reshape(1, F))
    return out


# speedup vs baseline: 1.0826x; 1.0000x over previous
import functools

import jax
import jax.numpy as jnp
from jax import lax
from jax.experimental import pallas as pl
from jax.experimental.pallas import tpu as pltpu
from jax.experimental.pallas import tpu_sc as plsc

N_NODES = 10000
F = 128
NC, NS, L = 2, 16, 16
NW = NC * NS
N_PAD = 10240
RPT = N_PAD // NS
CHUNK = 64
DEG_W = 16
ZB = 16
IBATCH = 40
DEPTH = 3
ROW_BLK = 2000


def _mesh():
    return plsc.VectorSubcoreMesh(core_axis_name="c", subcore_axis_name="s")


def _deg_body(nch_w, dst_hbm, degp_hbm, ibuf, hist, sumb, res, slab):
    c = lax.axis_index("c")
    s = lax.axis_index("s")
    wid = c * NS + s
    w0 = wid * nch_w
    zeros16 = jnp.zeros((L,), jnp.float32)
    ones16 = jnp.ones((L,), jnp.float32)

    def zfill(i, carry):
        hist[pl.ds(i * L, L)] = zeros16
        return carry

    lax.fori_loop(0, N_PAD // L, zfill, 0)

    nb = nch_w // IBATCH
    for b in range(nb):
        pltpu.sync_copy(dst_hbm.at[pl.ds(w0 + b * IBATCH, IBATCH)], ibuf)
        for j in range(IBATCH):
            for k in range(CHUNK // L):
                idx = ibuf[j, pl.ds(k * L, L)]
                plsc.addupdate_scatter(hist, [idx], ones16)

    pltpu.sync_copy(hist, slab.at[s])
    plsc.subcore_barrier()
    pltpu.sync_copy(slab.at[:, pl.ds(s * RPT, RPT)], sumb)

    def comb(k, carry):
        acc = sumb[0, pl.ds(k * L, L)]
        for t in range(1, NS):
            acc = acc + sumb[t, pl.ds(k * L, L)]
        idx = lax.iota(jnp.int32, L) * DEG_W + k * (L * DEG_W)
        plsc.store_scatter(res, [idx], acc)
        return carry

    lax.fori_loop(0, RPT // L, comb, 0)
    pltpu.sync_copy(res, degp_hbm.at[c, pl.ds(s * RPT * DEG_W, RPT * DEG_W)])


def _deg_call(eidx, nch_w):
    k = pl.kernel(
        functools.partial(_deg_body, nch_w),
        out_type=jax.ShapeDtypeStruct((NC, N_PAD * DEG_W), jnp.float32),
        mesh=_mesh(),
        compiler_params=pltpu.CompilerParams(needs_layout_passes=False),
        scratch_types=[
            pltpu.VMEM((IBATCH, CHUNK), jnp.int32),
            pltpu.VMEM((N_PAD,), jnp.float32),
            pltpu.VMEM((NS, RPT), jnp.float32),
            pltpu.VMEM((RPT * DEG_W,), jnp.float32),
            pltpu.VMEM_SHARED((NS, N_PAD), jnp.float32),
        ],
    )
    return k(eidx).reshape(NC, N_PAD, DEG_W)


def _scat_body(nch_w, hs_hbm, src_hbm, dst_hbm, part_hbm, ibs, ibd, rows0,
               rows1, rows2, stage, acc, gsem0, gsem1, gsem2, ssem0, ssem1,
               ssem2, isem):
    c = lax.axis_index("c")
    s = lax.axis_index("s")
    wid = c * NS + s
    w0 = wid * nch_w
    zeros16 = jnp.zeros((L,), jnp.float32)

    def zero(i, carry):
        for j in range(F // L):
            stage[i, pl.ds(j * L, L)] = zeros16
        return carry

    lax.fori_loop(0, ZB, zero, 0)
    row0 = s * RPT

    def zcopy(k, carry):
        pltpu.sync_copy(stage, acc.at[pl.ds(row0 + k * ZB, ZB)])
        return carry

    lax.fori_loop(0, RPT // ZB, zcopy, 0)
    plsc.subcore_barrier()

    rows = [rows0, rows1, rows2]
    gsem = [gsem0, gsem1, gsem2]
    ssem = [ssem0, ssem1, ssem2]
    nb = nch_w // IBATCH
    total = nb * IBATCH

    pltpu.sync_copy(src_hbm.at[pl.ds(w0, IBATCH)], ibs.at[0])
    pltpu.sync_copy(dst_hbm.at[pl.ds(w0, IBATCH)], ibd.at[0])
    gathers = [None] * DEPTH
    scats = [None] * DEPTH
    for g0 in range(min(DEPTH - 1, total)):
        gathers[g0] = pltpu.async_copy(
            hs_hbm.at[ibs.at[0, g0]], rows[g0], gsem[g0])
    iload_s = iload_d = None
    for g in range(total):
        b = g // IBATCH
        if g % IBATCH == 0 and b + 1 < nb:
            iload_s = pltpu.async_copy(
                src_hbm.at[pl.ds(w0 + (b + 1) * IBATCH, IBATCH)],
                ibs.at[(b + 1) % 2], isem)
            iload_d = pltpu.async_copy(
                dst_hbm.at[pl.ds(w0 + (b + 1) * IBATCH, IBATCH)],
                ibd.at[(b + 1) % 2], isem)
        nxt = g + DEPTH - 1
        if nxt < total:
            nbt, nj = divmod(nxt, IBATCH)
            if nj == 0:
                iload_s.wait()
                iload_d.wait()
            if scats[nxt % DEPTH] is not None:
                scats[nxt % DEPTH].wait()
                scats[nxt % DEPTH] = None
            gathers[nxt % DEPTH] = pltpu.async_copy(
                hs_hbm.at[ibs.at[nbt % 2, nj]], rows[nxt % DEPTH],
                gsem[nxt % DEPTH])
        gathers[g % DEPTH].wait()
        scats[g % DEPTH] = pltpu.async_copy(
            rows[g % DEPTH], acc.at[ibd.at[b % 2, g % IBATCH]],
            ssem[g % DEPTH], add=True)

    for p in range(DEPTH):
        if scats[p] is not None:
            scats[p].wait()

    plsc.subcore_barrier()

    def wcopy(k, carry):
        pltpu.sync_copy(acc.at[pl.ds(row0 + k * ZB, ZB)], stage)
        pltpu.sync_copy(stage, part_hbm.at[c, pl.ds(row0 + k * ZB, ZB)])
        return carry

    lax.fori_loop(0, RPT // ZB, wcopy, 0)


def _scat_call(hs, src_r, dst_r, nch_w):
    k = pl.kernel(
        functools.partial(_scat_body, nch_w),
        out_type=jax.ShapeDtypeStruct((NC, N_PAD, F), jnp.float32),
        mesh=_mesh(),
        scratch_types=[
            pltpu.VMEM((2, IBATCH, CHUNK), jnp.int32),
            pltpu.VMEM((2, IBATCH, CHUNK), jnp.int32),
            pltpu.VMEM((CHUNK, F), jnp.float32),
            pltpu.VMEM((CHUNK, F), jnp.float32),
            pltpu.VMEM((CHUNK, F), jnp.float32),
            pltpu.VMEM((ZB, F), jnp.float32),
            pltpu.VMEM_SHARED((N_PAD, F), jnp.float32),
            pltpu.SemaphoreType.DMA,
            pltpu.SemaphoreType.DMA,
            pltpu.SemaphoreType.DMA,
            pltpu.SemaphoreType.DMA,
            pltpu.SemaphoreType.DMA,
            pltpu.SemaphoreType.DMA,
            pltpu.SemaphoreType.DMA,
        ],
    )
    return k(hs, src_r, dst_r)


def _mm_body(x_ref, w_ref, degp_ref, hs_ref):
    deg = degp_ref[0, :, 0:1] + degp_ref[1, :, 0:1] + 1.0
    dis = lax.rsqrt(deg)
    h = jnp.dot(x_ref[...], w_ref[...], preferred_element_type=jnp.float32)
    hs_ref[...] = h * dis


def _mm_call(x, W, degp):
    grid = (N_NODES // ROW_BLK,)
    return pl.pallas_call(
        _mm_body,
        grid=grid,
        in_specs=[
            pl.BlockSpec((ROW_BLK, F), lambda i: (i, 0)),
            pl.BlockSpec((F, F), lambda i: (0, 0)),
            pl.BlockSpec((NC, ROW_BLK, DEG_W), lambda i: (0, i, 0)),
        ],
        out_specs=pl.BlockSpec((ROW_BLK, F), lambda i: (i, 0)),
        out_shape=jax.ShapeDtypeStruct((N_PAD, F), jnp.float32),
    )(x, W, degp)


def _fin_body(part_ref, hs_ref, degp_ref, b_ref, out_ref):
    deg = degp_ref[0, :, 0:1] + degp_ref[1, :, 0:1] + 1.0
    dis = lax.rsqrt(deg)
    out_ref[...] = dis * (part_ref[0] + part_ref[1] + hs_ref[...]) + b_ref[...]


def _fin_call(part, hs, degp, b2d):
    grid = (N_NODES // ROW_BLK,)
    return pl.pallas_call(
        _fin_body,
        grid=grid,
        in_specs=[
            pl.BlockSpec((NC, ROW_BLK, F), lambda i: (0, i, 0)),
            pl.BlockSpec((ROW_BLK, F), lambda i: (i, 0)),
            pl.BlockSpec((NC, ROW_BLK, DEG_W), lambda i: (0, i, 0)),
            pl.BlockSpec((1, F), lambda i: (0, 0)),
        ],
        out_specs=pl.BlockSpec((ROW_BLK, F), lambda i: (i, 0)),
        out_shape=jax.ShapeDtypeStruct((N_NODES, F), jnp.float32),
    )(part, hs, degp, b2d)


def kernel(x, edge_index, W, b):
    n, f_in = x.shape
    e = edge_index.shape[1]
    assert n == N_NODES and f_in == F

    nch = -(-e // CHUNK)
    nch_w = -(-nch // NW)
    nch_w = -(-nch_w // IBATCH) * IBATCH
    e_pad = nch_w * NW * CHUNK
    fill = N_NODES + (jnp.arange(e_pad - e, dtype=jnp.int32)
                      % (N_PAD - N_NODES))
    src_r = jnp.concatenate([edge_index[0].astype(jnp.int32), fill]
                            ).reshape(nch_w * NW, CHUNK)
    dst_r = jnp.concatenate([edge_index[1].astype(jnp.int32), fill]
                            ).reshape(nch_w * NW, CHUNK)

    degp = _deg_call(dst_r, nch_w)
    hs = _mm_call(x, W, degp)
    part = _scat_call(hs, src_r, dst_r, nch_w)
    out = _fin_call(part, hs, degp, b.reshape(1, F))
    return out
